# baseline (speedup 1.0000x reference)
import jax
import jax.numpy as jnp
from jax import lax
from jax.experimental import pallas as pl
from jax.experimental.pallas import tpu as pltpu
from jax.experimental.pallas import tpu_sc as plsc

B = 32
L = 2048
V = 100000
D = 128
C = 3
H = 224
P = 16
HP = H // P
NP = HP * HP
SEQ = L + NP + 1
EPS = 1e-6

NC = 2
NS = 16
NW = NC * NS

LHALF = L // 2
ROWS_PER_W = (B * LHALF) // NW
WCHUNK = 128
NWCHUNK = ROWS_PER_W // WCHUNK

NP1B = (NP + 1) * B
CPP = C * P * P


def _sc_words_body(table_hbm, idx_hbm, we_out,
                   idx_v, rows0, rows1, sem0, sem1, ssem0, ssem1):
    wid = lax.axis_index("s") * NC + lax.axis_index("c")
    base = wid * ROWS_PER_W
    pltpu.sync_copy(idx_hbm.at[pl.ds(base, ROWS_PER_W)], idx_v)
    bufs = (rows0, rows1)
    sems = (sem0, sem1)
    ssems = (ssem0, ssem1)
    cps = [None, None]
    scp = [None, None]
    cps[0] = pltpu.async_copy(
        table_hbm.at[idx_v.at[pl.ds(0, WCHUNK)]], rows0, sem0)
    for k in range(NWCHUNK):
        cur = k % 2
        nxt = (k + 1) % 2
        cps[cur].wait()
        scp[cur] = pltpu.async_copy(
            bufs[cur], we_out.at[pl.ds(base + k * WCHUNK, WCHUNK)], ssems[cur])
        if k + 1 < NWCHUNK:
            if scp[nxt] is not None:
                scp[nxt].wait()
            cps[nxt] = pltpu.async_copy(
                table_hbm.at[idx_v.at[pl.ds((k + 1) * WCHUNK, WCHUNK)]],
                bufs[nxt], sems[nxt])
    scp[0].wait()
    scp[1].wait()


def _sc_words(word_emb, tokens_lb):
    mesh = plsc.VectorSubcoreMesh(core_axis_name="c", subcore_axis_name="s")
    return pl.kernel(
        _sc_words_body,
        out_type=jax.ShapeDtypeStruct((B * LHALF, D), jnp.float32),
        mesh=mesh,
        scratch_types=[
            pltpu.VMEM((ROWS_PER_W,), jnp.int32),
            pltpu.VMEM((WCHUNK, D), jnp.float32),
            pltpu.VMEM((WCHUNK, D), jnp.float32),
            pltpu.SemaphoreType.DMA,
            pltpu.SemaphoreType.DMA,
            pltpu.SemaphoreType.DMA,
            pltpu.SemaphoreType.DMA,
        ],
    )(word_emb, tokens_lb)


def _sc_patches_body(img_hbm, pat_out,
                     slab0, slab1, oslab0, oslab1,
                     lsem0, lsem1, osem0, osem1):
    wid = lax.axis_index("s") * NC + lax.axis_index("c")
    b = wid

    def load_slab(r, sv, sem):
        return [pltpu.async_copy(img_hbm.at[b, c, pl.ds(P * r, P), :],
                                 sv.at[pl.ds(P * c, P)], sem)
                for c in range(C)]

    def fill(ov, sv):
        for q in range(HP):
            for ci in range(C * P):
                ov[q, pl.ds(P * ci, P)] = sv[ci, pl.ds(P * q, P)]

    def store_rows(ov, r, sem):
        return [pltpu.async_copy(ov.at[q], pat_out.at[B + (r * HP + q) * B + b],
                                 sem)
                for q in range(HP)]

    def slab_pair(t, carry):
        ra = 2 * t
        rb = ra + 1
        la = load_slab(ra, slab0, lsem0)
        lb = load_slab(rb, slab1, lsem1)
        for cp in la:
            cp.wait()
        fill(oslab0, slab0)
        oa = store_rows(oslab0, ra, osem0)
        for cp in lb:
            cp.wait()
        fill(oslab1, slab1)
        ob = store_rows(oslab1, rb, osem1)
        for cp in oa:
            cp.wait()
        for cp in ob:
            cp.wait()
        return carry

    lax.fori_loop(0, HP // 2, slab_pair, 0)


def _sc_patches(image):
    mesh = plsc.VectorSubcoreMesh(core_axis_name="c", subcore_axis_name="s")
    return pl.kernel(
        _sc_patches_body,
        out_type=jax.ShapeDtypeStruct((NP1B, CPP), jnp.float32),
        mesh=mesh,
        scratch_types=[
            pltpu.VMEM((C * P, H), jnp.float32),
            pltpu.VMEM((C * P, H), jnp.float32),
            pltpu.VMEM((HP, CPP), jnp.float32),
            pltpu.VMEM((HP, CPP), jnp.float32),
            pltpu.SemaphoreType.DMA,
            pltpu.SemaphoreType.DMA,
            pltpu.SemaphoreType.DMA,
            pltpu.SemaphoreType.DMA,
        ],
    )(image)


def _ln(x, g, b):
    j = jnp.full((D, D), 1.0 / D, dtype=jnp.float32)
    m = jnp.dot(x, j, preferred_element_type=jnp.float32)
    m2 = jnp.dot(x * x, j, preferred_element_type=jnp.float32)
    return g * ((x - m) * lax.rsqrt(m2 - m * m + EPS)) + b


SCHUNK = 128
NLC = L // SCHUNK
NVC = 2
PBLK = SCHUNK * B


def _tc_lang_body(params_ref, we_ref, pos_ref, out_ref):
    x = we_ref[...] + pos_ref[...][:, None, :]
    x2 = x.reshape(PBLK, D)
    z = _ln(_ln(x2, params_ref[0], params_ref[1]),
            params_ref[4], params_ref[5])
    out_ref[...] = z.reshape(SCHUNK, B, D)


def _tc_lang2_body(prev_ref, params_ref, we_ref, pos_ref, out_ref):
    del prev_ref
    _tc_lang_body(params_ref, we_ref, pos_ref, out_ref)


def _tc_vis_body(prev_ref, params_ref, pat_ref, w_ref, posv_ref, out_ref):
    del prev_ref
    c = pl.program_id(0)
    xb = pat_ref[...]
    proj = jnp.dot(xb, w_ref[...], preferred_element_type=jnp.float32)
    proj = proj + params_ref[6]
    proj3 = proj.reshape(SCHUNK, B, D)
    row0 = jax.lax.broadcasted_iota(jnp.int32, (SCHUNK, B, D), 0) == 0
    base3 = jnp.where((c == 0) & row0, params_ref[7], proj3)
    y3 = base3 + posv_ref[...][:, None, :]
    z = _ln(_ln(y3.reshape(PBLK, D), params_ref[2], params_ref[3]),
            params_ref[4], params_ref[5])
    out_ref[...] = z.reshape(SCHUNK, B, D)


def kernel(tokens, seg, image, word_emb, pos_emb_l, ln_l_g, ln_l_b, cls_token,
           conv_w, conv_b, pos_emb_v, ln_v_g, ln_v_b, ln_g, ln_b):
    del seg
    tokens_lb = tokens.T.reshape(L * B)

    we_a = _sc_words(word_emb, lax.slice(tokens_lb, (0,), (B * LHALF,)))
    we_b = _sc_words(word_emb, lax.slice(tokens_lb, (B * LHALF,), (B * L,)))
    we_a = we_a.reshape(LHALF, B, D)
    we_b = we_b.reshape(LHALF, B, D)
    pat = _sc_patches(image)

    wt = conv_w.reshape(D, CPP).T

    params = jnp.stack([
        ln_l_g, ln_l_b, ln_v_g, ln_v_b, ln_g, ln_b, conv_b,
        cls_token.reshape(D),
    ])

    NLH = NLC // 2
    out_a = pl.pallas_call(
        _tc_lang_body,
        out_shape=jax.ShapeDtypeStruct((SEQ, B, D), jnp.float32),
        grid=(NLH,),
        in_specs=[
            pl.BlockSpec((8, D), lambda c: (0, 0)),
            pl.BlockSpec((SCHUNK, B, D), lambda c: (c, 0, 0)),
            pl.BlockSpec((SCHUNK, D), lambda c: (c, 0)),
        ],
        out_specs=pl.BlockSpec((SCHUNK, B, D), lambda c: (c, 0, 0)),
    )(params, we_a, pos_emb_l)

    out_l = pl.pallas_call(
        _tc_lang2_body,
        out_shape=jax.ShapeDtypeStruct((SEQ, B, D), jnp.float32),
        grid=(NLH,),
        in_specs=[
            pl.BlockSpec(memory_space=pltpu.MemorySpace.HBM),
            pl.BlockSpec((8, D), lambda c: (0, 0)),
            pl.BlockSpec((SCHUNK, B, D), lambda c: (c, 0, 0)),
            pl.BlockSpec((SCHUNK, D), lambda c: (NLH + c, 0)),
        ],
        out_specs=pl.BlockSpec((SCHUNK, B, D), lambda c: (NLH + c, 0, 0)),
        input_output_aliases={0: 0},
    )(out_a, params, we_b, pos_emb_l)

    out_t = pl.pallas_call(
        _tc_vis_body,
        out_shape=jax.ShapeDtypeStruct((SEQ, B, D), jnp.float32),
        grid=(NVC,),
        in_specs=[
            pl.BlockSpec(memory_space=pltpu.MemorySpace.HBM),
            pl.BlockSpec((8, D), lambda c: (0, 0)),
            pl.BlockSpec((PBLK, CPP), lambda c: (c, 0)),
            pl.BlockSpec((CPP, D), lambda c: (0, 0)),
            pl.BlockSpec((SCHUNK, D), lambda c: (c, 0)),
        ],
        out_specs=pl.BlockSpec((SCHUNK, B, D), lambda c: (NLC + c, 0, 0)),
        input_output_aliases={0: 0},
    )(out_l, params, pat, wt, pos_emb_v)
    return out_t.transpose(1, 0, 2)


# speedup vs baseline: 1.0891x; 1.0891x over previous
import jax
import jax.numpy as jnp
from jax import lax
from jax.experimental import pallas as pl
from jax.experimental.pallas import tpu as pltpu
from jax.experimental.pallas import tpu_sc as plsc

B = 32
L = 2048
V = 100000
D = 128
C = 3
H = 224
P = 16
HP = H // P
NP = HP * HP
SEQ = L + NP + 1
EPS = 1e-6

NC = 2
NS = 16
NW = NC * NS

ROWS_PER_W = (B * L) // NW
WCHUNK = 128
NWCHUNK = ROWS_PER_W // WCHUNK

NP1B = (NP + 1) * B
CPP = C * P * P


def _sc_words_body(table_hbm, idx_hbm, we_out,
                   idx_v, rows0, rows1, sem0, sem1, ssem0, ssem1):
    wid = lax.axis_index("s") * NC + lax.axis_index("c")
    base = wid * ROWS_PER_W
    pltpu.sync_copy(idx_hbm.at[pl.ds(base, ROWS_PER_W)], idx_v)
    bufs = (rows0, rows1)
    sems = (sem0, sem1)
    ssems = (ssem0, ssem1)
    cps = [None, None]
    scp = [None, None]
    cps[0] = pltpu.async_copy(
        table_hbm.at[idx_v.at[pl.ds(0, WCHUNK)]], rows0, sem0)
    for k in range(NWCHUNK):
        cur = k % 2
        nxt = (k + 1) % 2
        cps[cur].wait()
        scp[cur] = pltpu.async_copy(
            bufs[cur], we_out.at[pl.ds(base + k * WCHUNK, WCHUNK)], ssems[cur])
        if k + 1 < NWCHUNK:
            if scp[nxt] is not None:
                scp[nxt].wait()
            cps[nxt] = pltpu.async_copy(
                table_hbm.at[idx_v.at[pl.ds((k + 1) * WCHUNK, WCHUNK)]],
                bufs[nxt], sems[nxt])
    scp[0].wait()
    scp[1].wait()


def _sc_words(word_emb, tokens_lb):
    mesh = plsc.VectorSubcoreMesh(core_axis_name="c", subcore_axis_name="s")
    return pl.kernel(
        _sc_words_body,
        out_type=jax.ShapeDtypeStruct((B * L, D), jnp.float32),
        mesh=mesh,
        scratch_types=[
            pltpu.VMEM((ROWS_PER_W,), jnp.int32),
            pltpu.VMEM((WCHUNK, D), jnp.float32),
            pltpu.VMEM((WCHUNK, D), jnp.float32),
            pltpu.SemaphoreType.DMA,
            pltpu.SemaphoreType.DMA,
            pltpu.SemaphoreType.DMA,
            pltpu.SemaphoreType.DMA,
        ],
    )(word_emb, tokens_lb)


def _sc_patches_body(img_hbm, pat_out,
                     slab0, slab1, oslab0, oslab1,
                     lsem0, lsem1, osem0, osem1):
    wid = lax.axis_index("s") * NC + lax.axis_index("c")
    b = wid

    def load_slab(r, sv, sem):
        return [pltpu.async_copy(img_hbm.at[b, c, pl.ds(P * r, P), :],
                                 sv.at[pl.ds(P * c, P)], sem)
                for c in range(C)]

    def fill(ov, sv):
        for q in range(HP):
            for ci in range(C * P):
                ov[q, pl.ds(P * ci, P)] = sv[ci, pl.ds(P * q, P)]

    def store_rows(ov, r, sem):
        return [pltpu.async_copy(ov.at[q], pat_out.at[B + (r * HP + q) * B + b],
                                 sem)
                for q in range(HP)]

    def slab_pair(t, carry):
        ra = 2 * t
        rb = ra + 1
        la = load_slab(ra, slab0, lsem0)
        lb = load_slab(rb, slab1, lsem1)
        for cp in la:
            cp.wait()
        fill(oslab0, slab0)
        oa = store_rows(oslab0, ra, osem0)
        for cp in lb:
            cp.wait()
        fill(oslab1, slab1)
        ob = store_rows(oslab1, rb, osem1)
        for cp in oa:
            cp.wait()
        for cp in ob:
            cp.wait()
        return carry

    lax.fori_loop(0, HP // 2, slab_pair, 0)


def _sc_patches(image):
    mesh = plsc.VectorSubcoreMesh(core_axis_name="c", subcore_axis_name="s")
    return pl.kernel(
        _sc_patches_body,
        out_type=jax.ShapeDtypeStruct((NP1B, CPP), jnp.float32),
        mesh=mesh,
        scratch_types=[
            pltpu.VMEM((C * P, H), jnp.float32),
            pltpu.VMEM((C * P, H), jnp.float32),
            pltpu.VMEM((HP, CPP), jnp.float32),
            pltpu.VMEM((HP, CPP), jnp.float32),
            pltpu.SemaphoreType.DMA,
            pltpu.SemaphoreType.DMA,
            pltpu.SemaphoreType.DMA,
            pltpu.SemaphoreType.DMA,
        ],
    )(image)


def _ln(x, g, b):
    j = jnp.full((D, D), 1.0 / D, dtype=jnp.float32)
    m = jnp.dot(x, j, preferred_element_type=jnp.float32)
    m2 = jnp.dot(x * x, j, preferred_element_type=jnp.float32)
    return g * ((x - m) * lax.rsqrt(m2 - m * m + EPS)) + b


SCHUNK = 128
NLC = L // SCHUNK
VCHUNK = 32
NVC = 7
PBLK = SCHUNK * B
VBLK = VCHUNK * B


def _tc_lang_body(params_ref, we_ref, pos_ref, out_ref):
    x = we_ref[...] + pos_ref[...][:, None, :]
    x2 = x.reshape(PBLK, D)
    z = _ln(_ln(x2, params_ref[0], params_ref[1]),
            params_ref[4], params_ref[5])
    out_ref[...] = z.reshape(SCHUNK, B, D)


def _tc_vis_body(prev_ref, params_ref, pat_ref, w_ref, posv_ref, out_ref):
    del prev_ref
    c = pl.program_id(0)
    xb = pat_ref[...]
    proj = jnp.dot(xb, w_ref[...], preferred_element_type=jnp.float32)
    proj = proj + params_ref[6]
    proj3 = proj.reshape(VCHUNK, B, D)
    row0 = jax.lax.broadcasted_iota(jnp.int32, (VCHUNK, B, D), 0) == 0
    base3 = jnp.where((c == 0) & row0, params_ref[7], proj3)
    y3 = base3 + posv_ref[...][:, None, :]
    z = _ln(_ln(y3.reshape(VBLK, D), params_ref[2], params_ref[3]),
            params_ref[4], params_ref[5])
    out_ref[...] = z.reshape(VCHUNK, B, D)


def kernel(tokens, seg, image, word_emb, pos_emb_l, ln_l_g, ln_l_b, cls_token,
           conv_w, conv_b, pos_emb_v, ln_v_g, ln_v_b, ln_g, ln_b):
    del seg
    tokens_lb = tokens.T.reshape(L * B)

    we_t = _sc_words(word_emb, tokens_lb).reshape(L, B, D)
    pat = _sc_patches(image)

    wt = conv_w.reshape(D, CPP).T

    params = jnp.stack([
        ln_l_g, ln_l_b, ln_v_g, ln_v_b, ln_g, ln_b, conv_b,
        cls_token.reshape(D),
    ])

    out_l = pl.pallas_call(
        _tc_lang_body,
        out_shape=jax.ShapeDtypeStruct((SEQ, B, D), jnp.float32),
        grid=(NLC,),
        in_specs=[
            pl.BlockSpec((8, D), lambda c: (0, 0)),
            pl.BlockSpec((SCHUNK, B, D), lambda c: (c, 0, 0)),
            pl.BlockSpec((SCHUNK, D), lambda c: (c, 0)),
        ],
        out_specs=pl.BlockSpec((SCHUNK, B, D), lambda c: (c, 0, 0)),
    )(params, we_t, pos_emb_l)

    out_t = pl.pallas_call(
        _tc_vis_body,
        out_shape=jax.ShapeDtypeStruct((SEQ, B, D), jnp.float32),
        grid=(NVC,),
        in_specs=[
            pl.BlockSpec(memory_space=pltpu.MemorySpace.HBM),
            pl.BlockSpec((8, D), lambda c: (0, 0)),
            pl.BlockSpec((VBLK, CPP), lambda c: (c, 0)),
            pl.BlockSpec((CPP, D), lambda c: (0, 0)),
            pl.BlockSpec((VCHUNK, D), lambda c: (c, 0)),
        ],
        out_specs=pl.BlockSpec((VCHUNK, B, D),
                               lambda c: (NLC * (SCHUNK // VCHUNK) + c, 0, 0)),
        input_output_aliases={0: 0},
    )(out_l, params, pat, wt, pos_emb_v)
    return out_t.transpose(1, 0, 2)
